# baseline (speedup 1.0000x reference)
import functools

import jax
import jax.numpy as jnp
from jax import lax
from jax.experimental import pallas as pl
from jax.experimental.pallas import tpu as pltpu
from jax.experimental.pallas import tpu_sc as plsc

N = 10000
D = 128
E = 320000
NG = 64
GRID = 8
NCLS = 10
GRID_MIN, GRID_MAX = -2.0, 2.0
EPS = 1e-5

NC, NS = 2, 16
NW = NC * NS
CHUNK = 96
NBUF = 3
N_PAD = 10240
ROWS_PER_TILE = N_PAD // NS
NCHUNK = NBUF * (-(-E // (NW * CHUNK * NBUF)))
EPW = NCHUNK * CHUNK
EP = EPW * NW
RBLK = 512
NBLK = N_PAD // RBLK


def _sc_agg_body(h_hbm, src_hbm, dst_hbm, out_hbm, *refs):
    srcv = refs[0:NBUF]
    dstv = refs[NBUF:2 * NBUF]
    rows = refs[2 * NBUF:3 * NBUF]
    acc = refs[3 * NBUF]
    sem = refs[3 * NBUF + 1:]
    c = lax.axis_index("c")
    s = lax.axis_index("s")
    wid = s * NC + c
    base = pl.multiple_of(wid * EPW, 8)

    def fetch_idx(j, t):
        off = pl.multiple_of(base + j * CHUNK, 8)
        pltpu.sync_copy(src_hbm.at[pl.ds(off, CHUNK)], srcv[t])
        pltpu.sync_copy(dst_hbm.at[pl.ds(off, CHUNK)], dstv[t])

    fetch_idx(0, 0)
    pltpu.async_copy(h_hbm.at[srcv[0]], rows[0], sem[0])
    for t in range(1, NBUF):
        fetch_idx(t, t)

    zbuf = rows[NBUF - 1]

    def zrow(i, carry):
        for j in range(D // 16):
            zbuf[i, pl.ds(j * 16, 16)] = jnp.zeros((16,), jnp.float32)
        return carry
    lax.fori_loop(0, CHUNK, zrow, 0)
    zoff = 0
    while zoff < ROWS_PER_TILE:
        zn = min(CHUNK, ROWS_PER_TILE - zoff)
        pltpu.sync_copy(zbuf.at[pl.ds(0, zn)],
                        acc.at[pl.ds(s * ROWS_PER_TILE + zoff, zn)])
        zoff += zn
    plsc.subcore_barrier()
    for t in range(1, NBUF):
        pltpu.async_copy(h_hbm.at[srcv[t]], rows[t], sem[t])

    def rot(k, carry):
        j0 = NBUF * k
        for t in range(NBUF):
            pltpu.make_async_copy(h_hbm.at[srcv[t]], rows[t], sem[t]).wait()
            pltpu.sync_copy(rows[t], acc.at[dstv[t]], add=True)
            fetch_idx(lax.rem(j0 + t + NBUF, NCHUNK), t)
            pltpu.async_copy(h_hbm.at[srcv[t]], rows[t], sem[t])
        return carry
    lax.fori_loop(0, NCHUNK // NBUF, rot, 0)
    for t in range(NBUF):
        pltpu.make_async_copy(h_hbm.at[srcv[t]], rows[t], sem[t]).wait()
    plsc.subcore_barrier()

    r0 = pl.multiple_of(s * ROWS_PER_TILE, 8)
    pltpu.sync_copy(acc.at[pl.ds(r0, ROWS_PER_TILE)],
                    out_hbm.at[c].at[pl.ds(r0, ROWS_PER_TILE)])


def _sc_aggregate(h, src_p, dst_p):
    mesh = plsc.VectorSubcoreMesh(core_axis_name="c", subcore_axis_name="s",
                                  num_cores=NC, num_subcores=NS)
    f = pl.kernel(
        _sc_agg_body,
        out_type=jax.ShapeDtypeStruct((NC, N_PAD, D), jnp.float32),
        mesh=mesh,
        scratch_types=(
            [pltpu.VMEM((CHUNK,), jnp.int32)] * (2 * NBUF)
            + [pltpu.VMEM((CHUNK, D), jnp.float32)] * NBUF
            + [pltpu.VMEM_SHARED((N_PAD, D), jnp.float32)]
            + [pltpu.SemaphoreType.DMA] * NBUF
        ),
    )
    return f(h, src_p, dst_p)


def _kan_sublayer(y, g, b, swT, sb, bwT, bb):
    mu = jnp.mean(y, axis=1, keepdims=True)
    d = y - mu
    var = jnp.mean(d * d, axis=1, keepdims=True)
    xn = d * lax.rsqrt(var + EPS) * g + b
    inv = (GRID - 1) / (GRID_MAX - GRID_MIN)
    step = (GRID_MAX - GRID_MIN) / (GRID - 1)
    parts = []
    for k in range(GRID):
        t = (xn - (GRID_MIN + k * step)) * inv
        parts.append(jnp.exp(-(t * t)))
    basis = jnp.concatenate(parts, axis=1)
    sil = xn * jax.nn.sigmoid(xn)
    return (jnp.dot(basis, swT, preferred_element_type=jnp.float32) + sb
            + jnp.dot(sil, bwT, preferred_element_type=jnp.float32) + bb)


def _conv_kan_body(h, p0, p1,
                   g1, b1, swT1, sb1, bwT1, bb1,
                   g2, b2, swT2, sb2, bwT2, bb2,
                   u_out, stats_out):
    i = pl.program_id(0)
    y = h[...] + p0[...] + p1[...]
    u = _kan_sublayer(y, g1[...], b1[...], swT1[...], sb1[...], bwT1[...], bb1[...])
    u = _kan_sublayer(u, g2[...], b2[...], swT2[...], sb2[...], bwT2[...], bb2[...])
    rid = i * RBLK + lax.broadcasted_iota(jnp.int32, (RBLK, 1), 0)
    u = jnp.where(rid < N, u, 0.0)
    u_out[...] = u
    st = jnp.concatenate([jnp.sum(u, axis=0, keepdims=True),
                          jnp.sum(u * u, axis=0, keepdims=True)], axis=0)

    @pl.when(i == 0)
    def _():
        stats_out[...] = st

    @pl.when(i > 0)
    def _():
        stats_out[...] = stats_out[...] + st


def _bn_affine(stats, g, b):
    mu = stats[0:1, :] * (1.0 / N)
    var = stats[1:2, :] * (1.0 / N) - mu * mu
    a = g * lax.rsqrt(var + EPS)
    c = b - mu * a
    return a, c


def _bn_apply_body(u, stats, g, b, h_out):
    i = pl.program_id(0)
    a, c = _bn_affine(stats[...], g[...], b[...])
    rid = i * RBLK + lax.broadcasted_iota(jnp.int32, (RBLK, 1), 0)
    h_out[...] = jnp.where(rid < N, u[...] * a + c, 0.0)


def _pool_kan_body(u, stats, g, b, batch3,
                   kg1, kb1, kswT1, ksb1, kbwT1, kbb1,
                   kg2, kb2, kswT2, ksb2, kbwT2, kbb2,
                   out, pooled_acc):
    i = pl.program_id(0)
    a, c = _bn_affine(stats[...], g[...], b[...])
    rid = i * RBLK + lax.broadcasted_iota(jnp.int32, (RBLK, 1), 0)
    hb = jnp.where(rid < N, u[...] * a + c, 0.0)
    gids = lax.broadcasted_iota(jnp.int32, (NG, RBLK), 0)
    bm = jnp.broadcast_to(batch3[0], (NG, RBLK))
    oh = (gids == bm).astype(jnp.float32)
    part = jnp.dot(oh, hb, preferred_element_type=jnp.float32)

    @pl.when(i == 0)
    def _():
        pooled_acc[...] = part

    @pl.when(i > 0)
    def _():
        pooled_acc[...] = pooled_acc[...] + part

    @pl.when(i == NBLK - 1)
    def _():
        pool = pooled_acc[...]
        z = _kan_sublayer(pool, kg1[...], kb1[...], kswT1[...], ksb1[...],
                          kbwT1[...], kbb1[...])
        z = _kan_sublayer(z, kg2[...], kb2[...], kswT2[...], ksb2[...],
                          kbwT2[...], kbb2[...])
        cid = lax.broadcasted_iota(jnp.int32, (NG, D), 1)
        zm = jnp.where(cid < NCLS, z, -1e30)
        m = jnp.max(zm, axis=1, keepdims=True)
        ex = jnp.exp(zm - m)
        out[...] = zm - m - jnp.log(jnp.sum(ex, axis=1, keepdims=True))


def _prep_sub(p, dout_pad=None):
    dout, dtot = p['sw'].shape
    din = dtot // GRID
    swT = p['sw'].reshape(dout, din, GRID).transpose(2, 1, 0).reshape(GRID * din, dout)
    bwT = p['bw'].T
    sb = p['sb'].reshape(1, dout)
    bb = p['bb'].reshape(1, dout)
    g = p['ln_g'].reshape(1, din)
    b = p['ln_b'].reshape(1, din)
    if dout_pad is not None and dout_pad != dout:
        swT = jnp.pad(swT, ((0, 0), (0, dout_pad - dout)))
        bwT = jnp.pad(bwT, ((0, 0), (0, dout_pad - dout)))
        sb = jnp.pad(sb, ((0, 0), (0, dout_pad - dout)))
        bb = jnp.pad(bb, ((0, 0), (0, dout_pad - dout)))
    return (g, b, swT, sb, bwT, bb)


def _wspecs(ws):
    return [pl.BlockSpec(w.shape, lambda i: (0,) * w.ndim) for w in ws]


def kernel(x, edge_index, batch, params):
    trash = N + jnp.arange(EP - E, dtype=jnp.int32) % (N_PAD - N)
    src_p = jnp.concatenate([edge_index[0].astype(jnp.int32), trash])
    dst_p = jnp.concatenate([edge_index[1].astype(jnp.int32), trash])
    h = jnp.zeros((N_PAD, D), jnp.float32).at[:N].set(x)
    batch3 = jnp.full((N_PAD,), NG, jnp.int32).at[:N].set(batch)
    batch3 = batch3.reshape(NBLK, 1, RBLK)

    row_spec = pl.BlockSpec((RBLK, D), lambda i: (i, 0))
    stats_spec = pl.BlockSpec((2, D), lambda i: (0, 0))
    vec_spec = pl.BlockSpec((1, D), lambda i: (0, 0))

    out = None
    for li in range(3):
        ws = (_prep_sub(params['convs'][li][0])
              + _prep_sub(params['convs'][li][1]))
        p = _sc_aggregate(h, src_p, dst_p)
        u, stats = pl.pallas_call(
            _conv_kan_body,
            grid=(NBLK,),
            in_specs=[row_spec, row_spec, row_spec] + _wspecs(ws),
            out_specs=[row_spec, stats_spec],
            out_shape=[jax.ShapeDtypeStruct((N_PAD, D), jnp.float32),
                       jax.ShapeDtypeStruct((2, D), jnp.float32)],
        )(h, p[0], p[1], *ws)
        bng = params['bn'][li]['g'].reshape(1, D)
        bnb = params['bn'][li]['b'].reshape(1, D)
        if li < 2:
            h = pl.pallas_call(
                _bn_apply_body,
                grid=(NBLK,),
                in_specs=[row_spec, stats_spec, vec_spec, vec_spec],
                out_specs=row_spec,
                out_shape=jax.ShapeDtypeStruct((N_PAD, D), jnp.float32),
            )(u, stats, bng, bnb)
        else:
            kw = (_prep_sub(params['kan'][0])
                  + _prep_sub(params['kan'][1], dout_pad=D))
            out = pl.pallas_call(
                _pool_kan_body,
                grid=(NBLK,),
                in_specs=([row_spec, stats_spec, vec_spec, vec_spec,
                           pl.BlockSpec((1, 1, RBLK), lambda i: (i, 0, 0))]
                          + _wspecs(kw)),
                out_specs=pl.BlockSpec((NG, D), lambda i: (0, 0)),
                out_shape=jax.ShapeDtypeStruct((NG, D), jnp.float32),
                scratch_shapes=[pltpu.VMEM((NG, D), jnp.float32)],
            )(u, stats, bng, bnb, batch3, *kw)
    return out[:, :NCLS]


# speedup vs baseline: 1.0811x; 1.0811x over previous
import functools

import jax
import jax.numpy as jnp
from jax import lax
from jax.experimental import pallas as pl
from jax.experimental.pallas import tpu as pltpu
from jax.experimental.pallas import tpu_sc as plsc

N = 10000
D = 128
E = 320000
NG = 64
GRID = 8
NCLS = 10
GRID_MIN, GRID_MAX = -2.0, 2.0
EPS = 1e-5

NC, NS = 2, 16
NW = NC * NS
CHUNK = 128
N_PAD = 10240
ROWS_PER_TILE = N_PAD // NS
NCHUNK = 2 * (-(-E // (NW * CHUNK * 2)))
EPW = NCHUNK * CHUNK
EP = EPW * NW
RBLK = 512
NBLK = N_PAD // RBLK


def _sc_agg_body(h_hbm, src_hbm, dst_hbm, out_hbm,
                 srcv0, dstv0, srcv1, dstv1, rows0, rows1, acc, sem0, sem1):
    c = lax.axis_index("c")
    s = lax.axis_index("s")
    wid = s * NC + c
    base = pl.multiple_of(wid * EPW, 8)

    def fetch_idx(j, sv, dv):
        off = pl.multiple_of(base + j * CHUNK, 8)
        pltpu.sync_copy(src_hbm.at[pl.ds(off, CHUNK)], sv)
        pltpu.sync_copy(dst_hbm.at[pl.ds(off, CHUNK)], dv)

    fetch_idx(0, srcv0, dstv0)
    pltpu.async_copy(h_hbm.at[srcv0], rows0, sem0)
    fetch_idx(1, srcv1, dstv1)

    def zrow(i, carry):
        for j in range(D // 16):
            rows1[i, pl.ds(j * 16, 16)] = jnp.zeros((16,), jnp.float32)
        return carry
    lax.fori_loop(0, CHUNK, zrow, 0)
    for k in range(ROWS_PER_TILE // CHUNK):
        pltpu.sync_copy(rows1, acc.at[pl.ds(s * ROWS_PER_TILE + k * CHUNK, CHUNK)])
    plsc.subcore_barrier()
    pltpu.async_copy(h_hbm.at[srcv1], rows1, sem1)

    def pair(k, carry):
        j0 = 2 * k
        pltpu.make_async_copy(h_hbm.at[srcv0], rows0, sem0).wait()
        pltpu.sync_copy(rows0, acc.at[dstv0], add=True)
        fetch_idx(lax.rem(j0 + 2, NCHUNK), srcv0, dstv0)
        pltpu.async_copy(h_hbm.at[srcv0], rows0, sem0)
        pltpu.make_async_copy(h_hbm.at[srcv1], rows1, sem1).wait()
        pltpu.sync_copy(rows1, acc.at[dstv1], add=True)
        fetch_idx(lax.rem(j0 + 3, NCHUNK), srcv1, dstv1)
        pltpu.async_copy(h_hbm.at[srcv1], rows1, sem1)
        return carry
    lax.fori_loop(0, NCHUNK // 2, pair, 0)
    pltpu.make_async_copy(h_hbm.at[srcv0], rows0, sem0).wait()
    pltpu.make_async_copy(h_hbm.at[srcv1], rows1, sem1).wait()
    plsc.subcore_barrier()

    r0 = pl.multiple_of(s * ROWS_PER_TILE, 8)
    pltpu.sync_copy(acc.at[pl.ds(r0, ROWS_PER_TILE)],
                    out_hbm.at[c].at[pl.ds(r0, ROWS_PER_TILE)])


def _sc_aggregate(h, src_p, dst_p):
    mesh = plsc.VectorSubcoreMesh(core_axis_name="c", subcore_axis_name="s",
                                  num_cores=NC, num_subcores=NS)
    f = pl.kernel(
        _sc_agg_body,
        out_type=jax.ShapeDtypeStruct((NC, N_PAD, D), jnp.float32),
        mesh=mesh,
        scratch_types=[
            pltpu.VMEM((CHUNK,), jnp.int32),
            pltpu.VMEM((CHUNK,), jnp.int32),
            pltpu.VMEM((CHUNK,), jnp.int32),
            pltpu.VMEM((CHUNK,), jnp.int32),
            pltpu.VMEM((CHUNK, D), jnp.float32),
            pltpu.VMEM((CHUNK, D), jnp.float32),
            pltpu.VMEM_SHARED((N_PAD, D), jnp.float32),
            pltpu.SemaphoreType.DMA,
            pltpu.SemaphoreType.DMA,
        ],
    )
    return f(h, src_p, dst_p)


def _kan_sublayer(y, g, b, swT, sb, bwT, bb):
    mu = jnp.mean(y, axis=1, keepdims=True)
    d = y - mu
    var = jnp.mean(d * d, axis=1, keepdims=True)
    xn = d * lax.rsqrt(var + EPS) * g + b
    inv = (GRID - 1) / (GRID_MAX - GRID_MIN)
    step = (GRID_MAX - GRID_MIN) / (GRID - 1)
    parts = []
    for k in range(GRID):
        t = (xn - (GRID_MIN + k * step)) * inv
        parts.append(jnp.exp(-(t * t)))
    basis = jnp.concatenate(parts, axis=1)
    sil = xn * jax.nn.sigmoid(xn)
    return (jnp.dot(basis, swT, preferred_element_type=jnp.float32) + sb
            + jnp.dot(sil, bwT, preferred_element_type=jnp.float32) + bb)


def _conv_kan_body(h, p0, p1,
                   g1, b1, swT1, sb1, bwT1, bb1,
                   g2, b2, swT2, sb2, bwT2, bb2,
                   u_out, stats_out):
    i = pl.program_id(0)
    y = h[...] + p0[...] + p1[...]
    u = _kan_sublayer(y, g1[...], b1[...], swT1[...], sb1[...], bwT1[...], bb1[...])
    u = _kan_sublayer(u, g2[...], b2[...], swT2[...], sb2[...], bwT2[...], bb2[...])
    rid = i * RBLK + lax.broadcasted_iota(jnp.int32, (RBLK, 1), 0)
    u = jnp.where(rid < N, u, 0.0)
    u_out[...] = u
    st = jnp.concatenate([jnp.sum(u, axis=0, keepdims=True),
                          jnp.sum(u * u, axis=0, keepdims=True)], axis=0)

    @pl.when(i == 0)
    def _():
        stats_out[...] = st

    @pl.when(i > 0)
    def _():
        stats_out[...] = stats_out[...] + st


def _bn_affine(stats, g, b):
    mu = stats[0:1, :] * (1.0 / N)
    var = stats[1:2, :] * (1.0 / N) - mu * mu
    a = g * lax.rsqrt(var + EPS)
    c = b - mu * a
    return a, c


def _bn_apply_body(u, stats, g, b, h_out):
    i = pl.program_id(0)
    a, c = _bn_affine(stats[...], g[...], b[...])
    rid = i * RBLK + lax.broadcasted_iota(jnp.int32, (RBLK, 1), 0)
    h_out[...] = jnp.where(rid < N, u[...] * a + c, 0.0)


def _pool_kan_body(u, stats, g, b, batch3,
                   kg1, kb1, kswT1, ksb1, kbwT1, kbb1,
                   kg2, kb2, kswT2, ksb2, kbwT2, kbb2,
                   out, pooled_acc):
    i = pl.program_id(0)
    a, c = _bn_affine(stats[...], g[...], b[...])
    rid = i * RBLK + lax.broadcasted_iota(jnp.int32, (RBLK, 1), 0)
    hb = jnp.where(rid < N, u[...] * a + c, 0.0)
    gids = lax.broadcasted_iota(jnp.int32, (NG, RBLK), 0)
    bm = jnp.broadcast_to(batch3[0], (NG, RBLK))
    oh = (gids == bm).astype(jnp.float32)
    part = jnp.dot(oh, hb, preferred_element_type=jnp.float32)

    @pl.when(i == 0)
    def _():
        pooled_acc[...] = part

    @pl.when(i > 0)
    def _():
        pooled_acc[...] = pooled_acc[...] + part

    @pl.when(i == NBLK - 1)
    def _():
        pool = pooled_acc[...]
        z = _kan_sublayer(pool, kg1[...], kb1[...], kswT1[...], ksb1[...],
                          kbwT1[...], kbb1[...])
        z = _kan_sublayer(z, kg2[...], kb2[...], kswT2[...], ksb2[...],
                          kbwT2[...], kbb2[...])
        cid = lax.broadcasted_iota(jnp.int32, (NG, D), 1)
        zm = jnp.where(cid < NCLS, z, -1e30)
        m = jnp.max(zm, axis=1, keepdims=True)
        ex = jnp.exp(zm - m)
        out[...] = zm - m - jnp.log(jnp.sum(ex, axis=1, keepdims=True))


def _prep_sub(p, dout_pad=None):
    dout, dtot = p['sw'].shape
    din = dtot // GRID
    swT = p['sw'].reshape(dout, din, GRID).transpose(2, 1, 0).reshape(GRID * din, dout)
    bwT = p['bw'].T
    sb = p['sb'].reshape(1, dout)
    bb = p['bb'].reshape(1, dout)
    g = p['ln_g'].reshape(1, din)
    b = p['ln_b'].reshape(1, din)
    if dout_pad is not None and dout_pad != dout:
        swT = jnp.pad(swT, ((0, 0), (0, dout_pad - dout)))
        bwT = jnp.pad(bwT, ((0, 0), (0, dout_pad - dout)))
        sb = jnp.pad(sb, ((0, 0), (0, dout_pad - dout)))
        bb = jnp.pad(bb, ((0, 0), (0, dout_pad - dout)))
    return (g, b, swT, sb, bwT, bb)


def _wspecs(ws):
    return [pl.BlockSpec(w.shape, lambda i: (0,) * w.ndim) for w in ws]


def kernel(x, edge_index, batch, params):
    trash = N + jnp.arange(EP - E, dtype=jnp.int32) % (N_PAD - N)
    src_p = jnp.concatenate([edge_index[0].astype(jnp.int32), trash])
    dst_p = jnp.concatenate([edge_index[1].astype(jnp.int32), trash])
    h = jnp.zeros((N_PAD, D), jnp.float32).at[:N].set(x)
    batch3 = jnp.full((N_PAD,), NG, jnp.int32).at[:N].set(batch)
    batch3 = batch3.reshape(NBLK, 1, RBLK)

    row_spec = pl.BlockSpec((RBLK, D), lambda i: (i, 0))
    stats_spec = pl.BlockSpec((2, D), lambda i: (0, 0))
    vec_spec = pl.BlockSpec((1, D), lambda i: (0, 0))

    out = None
    for li in range(3):
        ws = (_prep_sub(params['convs'][li][0])
              + _prep_sub(params['convs'][li][1]))
        p = _sc_aggregate(h, src_p, dst_p)
        u, stats = pl.pallas_call(
            _conv_kan_body,
            grid=(NBLK,),
            in_specs=[row_spec, row_spec, row_spec] + _wspecs(ws),
            out_specs=[row_spec, stats_spec],
            out_shape=[jax.ShapeDtypeStruct((N_PAD, D), jnp.float32),
                       jax.ShapeDtypeStruct((2, D), jnp.float32)],
        )(h, p[0], p[1], *ws)
        bng = params['bn'][li]['g'].reshape(1, D)
        bnb = params['bn'][li]['b'].reshape(1, D)
        if li < 2:
            h = pl.pallas_call(
                _bn_apply_body,
                grid=(NBLK,),
                in_specs=[row_spec, stats_spec, vec_spec, vec_spec],
                out_specs=row_spec,
                out_shape=jax.ShapeDtypeStruct((N_PAD, D), jnp.float32),
            )(u, stats, bng, bnb)
        else:
            kw = (_prep_sub(params['kan'][0])
                  + _prep_sub(params['kan'][1], dout_pad=D))
            out = pl.pallas_call(
                _pool_kan_body,
                grid=(NBLK,),
                in_specs=([row_spec, stats_spec, vec_spec, vec_spec,
                           pl.BlockSpec((1, 1, RBLK), lambda i: (i, 0, 0))]
                          + _wspecs(kw)),
                out_specs=pl.BlockSpec((NG, D), lambda i: (0, 0)),
                out_shape=jax.ShapeDtypeStruct((NG, D), jnp.float32),
                scratch_shapes=[pltpu.VMEM((NG, D), jnp.float32)],
            )(u, stats, bng, bnb, batch3, *kw)
    return out[:, :NCLS]
